# Initial kernel scaffold; baseline (speedup 1.0000x reference)
import jax
import jax.numpy as jnp
from jax.experimental import pallas as pl


def kernel(input_index, embeds):
    raise NotImplementedError("write your pallas kernel here")


# speedup vs baseline: 1.0946x; 1.0946x over previous
import functools

import jax
import jax.numpy as jnp
from jax import lax
from jax.experimental import pallas as pl
from jax.experimental.pallas import tpu as pltpu
from jax.experimental.pallas import tpu_sc as plsc

NUM_EMB = 1_000_000
D = 32
B = 16384 * 50
NC, NS = 2, 16
NW = NC * NS
BPW = B // NW
SUBLEN = 128
CHUNK = 1024
SUB = CHUNK // SUBLEN
NCHUNK = BPW // CHUNK


def _body(idx_hbm, table_hbm, out_hbm, idx_v, rows_v, sem):
    wid = lax.axis_index("s") * NC + lax.axis_index("c")
    row0 = wid * (BPW // SUBLEN)

    @pl.loop(0, NCHUNK)
    def _chunk(i):
        irow = row0 + i * SUB
        pltpu.sync_copy(idx_hbm.at[pl.ds(irow, SUB)], idx_v)
        cps = [
            pltpu.async_copy(
                table_hbm.at[idx_v.at[j]],
                rows_v.at[pl.ds(j * SUBLEN, SUBLEN)],
                sem,
            )
            for j in range(SUB)
        ]
        for cp in cps:
            cp.wait()
        pltpu.sync_copy(rows_v, out_hbm.at[pl.ds(irow * SUBLEN, CHUNK)])


@jax.jit
def _gather(idx2, embeds):
    k = functools.partial(
        pl.kernel,
        out_type=jax.ShapeDtypeStruct((B, D), jnp.float32),
        mesh=plsc.VectorSubcoreMesh(core_axis_name="c", subcore_axis_name="s"),
        scratch_types=[
            pltpu.VMEM((SUB, SUBLEN), jnp.int32),
            pltpu.VMEM((CHUNK, D), jnp.float32),
            pltpu.SemaphoreType.DMA,
        ],
        compiler_params=pltpu.CompilerParams(use_tc_tiling_on_sc=False),
    )(_body)
    return k(idx2, embeds)


def kernel(input_index, embeds):
    idx2 = input_index.reshape(B // SUBLEN, SUBLEN).astype(jnp.int32)
    out = _gather(idx2, embeds)
    return out.reshape(input_index.shape + (D,))


# speedup vs baseline: 1.1103x; 1.0144x over previous
import functools

import jax
import jax.numpy as jnp
from jax import lax
from jax.experimental import pallas as pl
from jax.experimental.pallas import tpu as pltpu
from jax.experimental.pallas import tpu_sc as plsc

NUM_EMB = 1_000_000
D = 32
B = 16384 * 50
NC, NS = 2, 16
NW = NC * NS
BPW = B // NW
SUBLEN = 128
CHUNK = 1280
SUB = CHUNK // SUBLEN
NCHUNK = BPW // CHUNK


def _body(idx_hbm, table_hbm, out_hbm, idx_v, rows_v,
          sem_g0, sem_g1, sem_w0, sem_w1):
    sem_g = (sem_g0, sem_g1)
    sem_w = (sem_w0, sem_w1)
    wid = lax.axis_index("s") * NC + lax.axis_index("c")
    row0 = wid * (BPW // SUBLEN)

    def load_idx(i, s):
        pltpu.sync_copy(idx_hbm.at[pl.ds(row0 + i * SUB, SUB)], idx_v.at[s])

    def gather_cp(s, j):
        return pltpu.make_async_copy(
            table_hbm.at[idx_v.at[s, j]],
            rows_v.at[s, pl.ds(j * SUBLEN, SUBLEN)],
            sem_g[s],
        )

    def write_cp(i, s):
        return pltpu.make_async_copy(
            rows_v.at[s],
            out_hbm.at[pl.ds((row0 + i * SUB) * SUBLEN, CHUNK)],
            sem_w[s],
        )

    def fire_gathers(s):
        for j in range(SUB):
            gather_cp(s, j).start()

    def wait_gathers(s):
        for j in range(SUB):
            gather_cp(s, j).wait()

    load_idx(0, 0)
    fire_gathers(0)

    @pl.loop(0, NCHUNK, step=2)
    def _outer(base):
        for b in range(2):
            i = base + b
            s, o = b, 1 - b

            @pl.when(i + 1 < NCHUNK)
            def _prefetch_idx():
                load_idx(i + 1, o)

            wait_gathers(s)

            @pl.when(i + 1 < NCHUNK)
            def _next_gathers():
                @pl.when(i >= 1)
                def _recycle():
                    write_cp(i - 1, o).wait()

                fire_gathers(o)

            write_cp(i, s).start()

    write_cp(NCHUNK - 2, 0).wait()
    write_cp(NCHUNK - 1, 1).wait()


@jax.jit
def _gather(idx2, embeds):
    k = functools.partial(
        pl.kernel,
        out_type=jax.ShapeDtypeStruct((B, D), jnp.float32),
        mesh=plsc.VectorSubcoreMesh(core_axis_name="c", subcore_axis_name="s"),
        scratch_types=[
            pltpu.VMEM((2, SUB, SUBLEN), jnp.int32),
            pltpu.VMEM((2, CHUNK, D), jnp.float32),
            pltpu.SemaphoreType.DMA,
            pltpu.SemaphoreType.DMA,
            pltpu.SemaphoreType.DMA,
            pltpu.SemaphoreType.DMA,
        ],
        compiler_params=pltpu.CompilerParams(use_tc_tiling_on_sc=False),
    )(_body)
    return k(idx2, embeds)


def kernel(input_index, embeds):
    idx2 = input_index.reshape(B // SUBLEN, SUBLEN).astype(jnp.int32)
    out = _gather(idx2, embeds)
    return out.reshape(input_index.shape + (D,))


# speedup vs baseline: 1.1132x; 1.0026x over previous
import functools

import jax
import jax.numpy as jnp
from jax import lax
from jax.experimental import pallas as pl
from jax.experimental.pallas import tpu as pltpu
from jax.experimental.pallas import tpu_sc as plsc

NUM_EMB = 1_000_000
D = 32
B = 16384 * 50
NC, NS = 2, 16
NW = NC * NS
BPW = B // NW
SUBLEN = 128
CHUNK = 1280
SUB = CHUNK // SUBLEN
NCHUNK = BPW // CHUNK


def _body(idx_hbm, table_hbm, out_hbm, idx_v, rows_v,
          sem_i, sem_g0, sem_g1, sem_w0, sem_w1):
    sem_g = (sem_g0, sem_g1)
    sem_w = (sem_w0, sem_w1)
    wid = lax.axis_index("s") * NC + lax.axis_index("c")
    row0 = wid * (BPW // SUBLEN)

    def idx_cp(i, si):
        return pltpu.make_async_copy(
            idx_hbm.at[pl.ds(row0 + i * SUB, SUB)], idx_v.at[si], sem_i)

    def gather_cp(si, sr, j):
        return pltpu.make_async_copy(
            table_hbm.at[idx_v.at[si, j]],
            rows_v.at[sr, pl.ds(j * SUBLEN, SUBLEN)],
            sem_g[sr],
        )

    def write_cp(i, sr):
        return pltpu.make_async_copy(
            rows_v.at[sr],
            out_hbm.at[pl.ds((row0 + i * SUB) * SUBLEN, CHUNK)],
            sem_w[sr],
        )

    def fire_gathers(si, sr):
        for j in range(SUB):
            gather_cp(si, sr, j).start()

    def wait_gathers(si, sr):
        for j in range(SUB):
            gather_cp(si, sr, j).wait()

    idx_cp(0, 0).start()
    idx_cp(1, 1).start()
    idx_cp(0, 0).wait()
    fire_gathers(0, 0)

    @pl.loop(0, NCHUNK, step=4)
    def _outer(base):
        for b in range(4):
            i = base + b

            @pl.when(i + 2 < NCHUNK)
            def _prefetch_idx():
                idx_cp(i + 2, (b + 2) % 4).start()

            @pl.when(i + 1 < NCHUNK)
            def _next_gathers():
                @pl.when(i >= 1)
                def _recycle():
                    write_cp(i - 1, (b + 1) % 2).wait()

                idx_cp(i + 1, (b + 1) % 4).wait()
                fire_gathers((b + 1) % 4, (b + 1) % 2)

            wait_gathers(b % 4, b % 2)

            write_cp(i, b % 2).start()

    write_cp(NCHUNK - 2, 0).wait()
    write_cp(NCHUNK - 1, 1).wait()


@jax.jit
def _gather(idx2, embeds):
    k = functools.partial(
        pl.kernel,
        out_type=jax.ShapeDtypeStruct((B, D), jnp.float32),
        mesh=plsc.VectorSubcoreMesh(core_axis_name="c", subcore_axis_name="s"),
        scratch_types=[
            pltpu.VMEM((4, SUB, SUBLEN), jnp.int32),
            pltpu.VMEM((2, CHUNK, D), jnp.float32),
            pltpu.SemaphoreType.DMA,
            pltpu.SemaphoreType.DMA,
            pltpu.SemaphoreType.DMA,
            pltpu.SemaphoreType.DMA,
            pltpu.SemaphoreType.DMA,
        ],
        compiler_params=pltpu.CompilerParams(use_tc_tiling_on_sc=False),
    )(_body)
    return k(idx2, embeds)


def kernel(input_index, embeds):
    idx2 = input_index.reshape(B // SUBLEN, SUBLEN).astype(jnp.int32)
    out = _gather(idx2, embeds)
    return out.reshape(input_index.shape + (D,))


# speedup vs baseline: 1.7290x; 1.5532x over previous
import functools

import jax
import jax.numpy as jnp
from jax import lax
from jax.experimental import pallas as pl
from jax.experimental.pallas import tpu as pltpu
from jax.experimental.pallas import tpu_sc as plsc

NUM_EMB = 1_000_000
D = 32
B = 16384 * 50
NC, NS = 2, 16
NW = NC * NS
BPW = B // NW
SUBLEN = 128
CHUNK = 1280
SUB = CHUNK // SUBLEN
NCHUNK = BPW // CHUNK


def _body(idx_hbm, table_hbm, out_hbm, idx_v, rows_v,
          sem_i, sem_g0, sem_g1, sem_w0, sem_w1):
    sem_g = (sem_g0, sem_g1)
    sem_w = (sem_w0, sem_w1)
    wid = lax.axis_index("s") * NC + lax.axis_index("c")
    row0 = wid * (BPW // SUBLEN)

    def idx_cp(i, si):
        return pltpu.make_async_copy(
            idx_hbm.at[pl.ds(row0 + i * SUB, SUB)], idx_v.at[si], sem_i)

    def gather_cp(si, sr, j):
        return pltpu.make_async_copy(
            table_hbm.at[idx_v.at[si, j]],
            rows_v.at[sr, pl.ds(j * SUBLEN, SUBLEN)],
            sem_g[sr],
        )

    def write_cp(i, sr):
        return pltpu.make_async_copy(
            rows_v.at[sr],
            out_hbm.at[pl.ds((row0 + i * SUB) * SUBLEN, CHUNK)],
            sem_w[sr],
        )

    def fire_gathers(si, sr):
        for j in range(SUB):
            gather_cp(si, sr, j).start()

    def wait_gathers(si, sr):
        for j in range(SUB):
            gather_cp(si, sr, j).wait()

    idx_cp(0, 0).start()
    idx_cp(1, 1).start()
    idx_cp(0, 0).wait()
    fire_gathers(0, 0)

    @pl.loop(0, NCHUNK, step=4)
    def _outer(base):
        for b in range(4):
            i = base + b

            @pl.when(i + 2 < NCHUNK)
            def _prefetch_idx():
                idx_cp(i + 2, (b + 2) % 4).start()

            @pl.when(i + 1 < NCHUNK)
            def _next_gathers():
                @pl.when(i >= 1)
                def _recycle():
                    write_cp(i - 1, (b + 1) % 2).wait()

                idx_cp(i + 1, (b + 1) % 4).wait()
                fire_gathers((b + 1) % 4, (b + 1) % 2)

            wait_gathers(b % 4, b % 2)

            write_cp(i, b % 2).start()

    write_cp(NCHUNK - 2, 0).wait()
    write_cp(NCHUNK - 1, 1).wait()


ROWS_PW = 16384 // NW
CB = 16
NFMT = ROWS_PW // CB
GROWS = CB * 50 * 32 // 128
WB = 4
NSUB = CB // WB


def _fmt_body(src_hbm, out_hbm, a_v, b_v, sem_r0, sem_r1, sem_w0, sem_w1):
    sem_r = (sem_r0, sem_r1)
    sem_w = (sem_w0, sem_w1)
    wid = lax.axis_index("s") * NC + lax.axis_index("c")
    r0 = wid * ROWS_PW
    g0 = wid * (ROWS_PW * 50 * 32 // 128)

    def read_cp(i, s):
        return pltpu.make_async_copy(
            src_hbm.at[pl.ds(g0 + i * GROWS, GROWS)], a_v.at[s], sem_r[s])

    def write_cp(m, t):
        return pltpu.make_async_copy(
            b_v.at[t], out_hbm.at[pl.ds(r0 + m * WB, WB)], sem_w[t])

    def relay(s, k, t):
        for v in range(WB * 50 * 32 // 16):
            x = a_v[s, 50 * k + v // 8, pl.ds((v % 8) * 16, 16)]
            b_v[t, v // 100, (v // 2) % 50, pl.ds((v % 2) * 16, 16)] = x

    read_cp(0, 0).start()

    @pl.loop(0, NFMT, step=2)
    def _outer(base):
        for b in range(2):
            i = base + b
            s, o = b, 1 - b

            @pl.when(i + 1 < NFMT)
            def _prefetch():
                read_cp(i + 1, o).start()

            read_cp(i, s).wait()

            for k in range(NSUB):
                m = i * NSUB + k
                t = k % 2
                if k < 2:
                    @pl.when(i >= 1)
                    def _recycle():
                        write_cp(m - 2, t).wait()
                else:
                    write_cp(m - 2, t).wait()
                relay(s, k, t)
                write_cp(m, t).start()

    write_cp(NFMT * NSUB - 2, 0).wait()
    write_cp(NFMT * NSUB - 1, 1).wait()


@jax.jit
def _gather(idx2, embeds):
    k = functools.partial(
        pl.kernel,
        out_type=jax.ShapeDtypeStruct((B, D), jnp.float32),
        mesh=plsc.VectorSubcoreMesh(core_axis_name="c", subcore_axis_name="s"),
        scratch_types=[
            pltpu.VMEM((4, SUB, SUBLEN), jnp.int32),
            pltpu.VMEM((2, CHUNK, D), jnp.float32),
            pltpu.SemaphoreType.DMA,
            pltpu.SemaphoreType.DMA,
            pltpu.SemaphoreType.DMA,
            pltpu.SemaphoreType.DMA,
            pltpu.SemaphoreType.DMA,
        ],
        compiler_params=pltpu.CompilerParams(use_tc_tiling_on_sc=False),
    )(_body)
    flat = k(idx2, embeds)

    fmt = functools.partial(
        pl.kernel,
        out_type=jax.ShapeDtypeStruct((16384, 50, D), jnp.float32),
        mesh=plsc.VectorSubcoreMesh(core_axis_name="c", subcore_axis_name="s"),
        scratch_types=[
            pltpu.VMEM((2, GROWS, 128), jnp.float32),
            pltpu.VMEM((2, WB, 50, D), jnp.float32),
            pltpu.SemaphoreType.DMA,
            pltpu.SemaphoreType.DMA,
            pltpu.SemaphoreType.DMA,
            pltpu.SemaphoreType.DMA,
        ],
        compiler_params=pltpu.CompilerParams(use_tc_tiling_on_sc=True),
    )(_fmt_body)
    return fmt(flat.reshape(B * D // 128, 128))


def kernel(input_index, embeds):
    idx2 = input_index.reshape(B // SUBLEN, SUBLEN).astype(jnp.int32)
    return _gather(idx2, embeds)


# speedup vs baseline: 1.7296x; 1.0003x over previous
import functools

import jax
import jax.numpy as jnp
from jax import lax
from jax.experimental import pallas as pl
from jax.experimental.pallas import tpu as pltpu
from jax.experimental.pallas import tpu_sc as plsc

NUM_EMB = 1_000_000
D = 32
B = 16384 * 50
NC, NS = 2, 16
NW = NC * NS
BPW = B // NW
SUBLEN = 128
CHUNK = 1280
SUB = CHUNK // SUBLEN
NCHUNK = BPW // CHUNK


def _body(idx_hbm, table_hbm, out_hbm, idx_v, rows_v,
          sem_i, sem_g0, sem_g1, sem_w0, sem_w1):
    sem_g = (sem_g0, sem_g1)
    sem_w = (sem_w0, sem_w1)
    wid = lax.axis_index("s") * NC + lax.axis_index("c")
    row0 = wid * (BPW // SUBLEN)

    def idx_cp(i, si):
        return pltpu.make_async_copy(
            idx_hbm.at[pl.ds((row0 + i * SUB) * SUBLEN, CHUNK)],
            idx_v.at[si], sem_i)

    def gather_cp(si, sr, j):
        return pltpu.make_async_copy(
            table_hbm.at[idx_v.at[si, pl.ds(j * SUBLEN, SUBLEN)]],
            rows_v.at[sr, pl.ds(j * SUBLEN, SUBLEN)],
            sem_g[sr],
        )

    def write_cp(i, sr):
        return pltpu.make_async_copy(
            rows_v.at[sr],
            out_hbm.at[pl.ds((row0 + i * SUB) * SUBLEN, CHUNK)],
            sem_w[sr],
        )

    def fire_gathers(si, sr):
        for j in range(SUB):
            gather_cp(si, sr, j).start()

    def wait_gathers(si, sr):
        for j in range(SUB):
            gather_cp(si, sr, j).wait()

    idx_cp(0, 0).start()
    idx_cp(1, 1).start()
    idx_cp(0, 0).wait()
    fire_gathers(0, 0)

    @pl.loop(0, NCHUNK, step=4)
    def _outer(base):
        for b in range(4):
            i = base + b

            @pl.when(i + 2 < NCHUNK)
            def _prefetch_idx():
                idx_cp(i + 2, (b + 2) % 4).start()

            @pl.when(i + 1 < NCHUNK)
            def _next_gathers():
                @pl.when(i >= 1)
                def _recycle():
                    write_cp(i - 1, (b + 1) % 2).wait()

                idx_cp(i + 1, (b + 1) % 4).wait()
                fire_gathers((b + 1) % 4, (b + 1) % 2)

            wait_gathers(b % 4, b % 2)

            write_cp(i, b % 2).start()

    write_cp(NCHUNK - 2, 0).wait()
    write_cp(NCHUNK - 1, 1).wait()


ROWS_PW = 16384 // NW
CB = 16
NFMT = ROWS_PW // CB
GROWS = CB * 50 * 32 // 128
WB = 4
NSUB = CB // WB


def _fmt_body(src_hbm, out_hbm, a_v, b_v, sem_r0, sem_r1, sem_w0, sem_w1):
    sem_r = (sem_r0, sem_r1)
    sem_w = (sem_w0, sem_w1)
    wid = lax.axis_index("s") * NC + lax.axis_index("c")
    r0 = wid * ROWS_PW
    g0 = wid * (ROWS_PW * 50 * 32 // 128)

    def read_cp(i, s):
        return pltpu.make_async_copy(
            src_hbm.at[pl.ds(g0 + i * GROWS, GROWS)], a_v.at[s], sem_r[s])

    def write_cp(m, t):
        return pltpu.make_async_copy(
            b_v.at[t], out_hbm.at[pl.ds(r0 + m * WB, WB)], sem_w[t])

    def relay(s, k, t):
        for v in range(WB * 50 * 32 // 16):
            x = a_v[s, 50 * k + v // 8, pl.ds((v % 8) * 16, 16)]
            b_v[t, v // 100, (v // 2) % 50, pl.ds((v % 2) * 16, 16)] = x

    read_cp(0, 0).start()

    @pl.loop(0, NFMT, step=2)
    def _outer(base):
        for b in range(2):
            i = base + b
            s, o = b, 1 - b

            @pl.when(i + 1 < NFMT)
            def _prefetch():
                read_cp(i + 1, o).start()

            read_cp(i, s).wait()

            for k in range(NSUB):
                m = i * NSUB + k
                t = k % 2
                if k < 2:
                    @pl.when(i >= 1)
                    def _recycle():
                        write_cp(m - 2, t).wait()
                else:
                    write_cp(m - 2, t).wait()
                relay(s, k, t)
                write_cp(m, t).start()

    write_cp(NFMT * NSUB - 2, 0).wait()
    write_cp(NFMT * NSUB - 1, 1).wait()


@jax.jit
def _gather(idx2, embeds):
    k = functools.partial(
        pl.kernel,
        out_type=jax.ShapeDtypeStruct((B, D), jnp.float32),
        mesh=plsc.VectorSubcoreMesh(core_axis_name="c", subcore_axis_name="s"),
        scratch_types=[
            pltpu.VMEM((4, CHUNK), jnp.int32),
            pltpu.VMEM((2, CHUNK, D), jnp.float32),
            pltpu.SemaphoreType.DMA,
            pltpu.SemaphoreType.DMA,
            pltpu.SemaphoreType.DMA,
            pltpu.SemaphoreType.DMA,
            pltpu.SemaphoreType.DMA,
        ],
        compiler_params=pltpu.CompilerParams(use_tc_tiling_on_sc=False),
    )(_body)
    flat = k(idx2, embeds)

    fmt = functools.partial(
        pl.kernel,
        out_type=jax.ShapeDtypeStruct((16384, 50, D), jnp.float32),
        mesh=plsc.VectorSubcoreMesh(core_axis_name="c", subcore_axis_name="s"),
        scratch_types=[
            pltpu.VMEM((2, GROWS, 128), jnp.float32),
            pltpu.VMEM((2, WB, 50, D), jnp.float32),
            pltpu.SemaphoreType.DMA,
            pltpu.SemaphoreType.DMA,
            pltpu.SemaphoreType.DMA,
            pltpu.SemaphoreType.DMA,
        ],
        compiler_params=pltpu.CompilerParams(use_tc_tiling_on_sc=True),
    )(_fmt_body)
    return fmt(flat.reshape(B * D // 128, 128))


def kernel(input_index, embeds):
    idx1 = input_index.reshape(B).astype(jnp.int32)
    return _gather(idx1, embeds)
